# baseline (speedup 1.0000x reference)
import functools

import jax
import jax.numpy as jnp
from jax import lax
from jax.experimental import pallas as pl
from jax.experimental.pallas import tpu as pltpu
from jax.experimental.pallas import tpu_sc as plsc

N = 10000
E = 320000
NFEAT = 128
NHID = 128
NCLASS = 40
D1 = 128

NC = 2
NS = 16
NW = NC * NS
EPW = E // NW
CH = 80
PC = 64
NPH = 2
NITER = PC * NPH
EPW_PAD = NITER * CH
N_PAD = 10240
ROWS_PER_TILE = N_PAD // NS
DUMMY_DST = N_PAD - 1


@functools.lru_cache(maxsize=None)
def _make_sc_agg(d):
  mesh = plsc.VectorSubcoreMesh(core_axis_name="c", subcore_axis_name="s")

  @functools.partial(
      pl.kernel,
      mesh=mesh,
      out_type=jax.ShapeDtypeStruct((NC, N_PAD, d), jnp.float32),
      scratch_types=[
          pltpu.VMEM((PC, CH), jnp.int32),
          pltpu.VMEM((PC, CH), jnp.int32),
          pltpu.VMEM((CH, d), jnp.float32),
          pltpu.VMEM((CH, d), jnp.float32),
          pltpu.VMEM_SHARED((N_PAD, d), jnp.float32),
          pltpu.SemaphoreType.DMA,
          pltpu.SemaphoreType.DMA,
      ],
  )
  def sc_agg(xl_hbm, src_hbm, dst_hbm, zero_hbm, out_hbm,
             src_v, dst_v, rows0_v, rows1_v, acc, sem0, sem1):
    c = lax.axis_index("c")
    s = lax.axis_index("s")
    wid = c * NS + s

    pltpu.sync_copy(zero_hbm, acc.at[pl.ds(s * ROWS_PER_TILE, ROWS_PER_TILE)])
    plsc.subcore_barrier()

    for p in range(NPH):
      pltpu.sync_copy(src_hbm.at[wid, pl.ds(p * PC, PC)], src_v)
      pltpu.sync_copy(dst_hbm.at[wid, pl.ds(p * PC, PC)], dst_v)

      def body(j, carry):
        pltpu.async_copy(xl_hbm.at[src_v.at[j]], rows0_v, sem0).wait()
        pltpu.sync_copy(rows0_v, acc.at[dst_v.at[j]], add=True)
        return carry

      lax.fori_loop(0, PC, body, 0, unroll=False)

    plsc.subcore_barrier()

    sl = pl.ds(s * ROWS_PER_TILE, ROWS_PER_TILE)
    pltpu.sync_copy(acc.at[sl], out_hbm.at[c, sl])

  return sc_agg


BN = 1000


def _layer_norm_block(xb, g, b, eps=1e-5):
  mu = jnp.mean(xb, axis=1, keepdims=True)
  var = jnp.mean((xb - mu) ** 2, axis=1, keepdims=True)
  return (xb - mu) / jnp.sqrt(var + eps) * g + b


def _tc_a_body(x_ref, g_ref, b_ref, wl_ref, bl_ref, wr_ref, xl_ref, xr_ref):
  h = _layer_norm_block(x_ref[...], g_ref[...], b_ref[...])
  xl_ref[...] = (jnp.dot(h, wl_ref[...], preferred_element_type=jnp.float32)
                 + bl_ref[...])
  xr_ref[...] = jnp.dot(h, wr_ref[...], preferred_element_type=jnp.float32)


def _tc_b_body(p0_ref, p1_ref, xr_ref, g_ref, b_ref, wl_ref, bl_ref, wr_ref,
               xl_ref, xr1_ref):
  s = p0_ref[...] + p1_ref[...] + xr_ref[...]
  nrm = jnp.sqrt(jnp.sum(s * s, axis=1, keepdims=True))
  h = s / jnp.maximum(nrm, 1e-12)
  h = _layer_norm_block(h, g_ref[...], b_ref[...])
  h = jnp.maximum(h, 0.0)
  xl_ref[...] = (jnp.dot(h, wl_ref[...], preferred_element_type=jnp.float32)
                 + bl_ref[...])
  xr1_ref[...] = jnp.dot(h, wr_ref[...], preferred_element_type=jnp.float32)


def _tc_c_body(p0_ref, p1_ref, xr_ref, o_ref):
  s = p0_ref[...] + p1_ref[...] + xr_ref[...]
  nrm = jnp.sqrt(jnp.sum(s * s, axis=1, keepdims=True))
  h = s / jnp.maximum(nrm, 1e-12)
  col = lax.broadcasted_iota(jnp.int32, h.shape, 1)
  valid = col < NCLASS
  hm = jnp.where(valid, h, -jnp.inf)
  m = jnp.max(hm, axis=1, keepdims=True)
  ex = jnp.where(valid, jnp.exp(h - m), 0.0)
  lse = jnp.log(jnp.sum(ex, axis=1, keepdims=True))
  o_ref[...] = h - m - lse


def _row_spec(bn, d):
  return pl.BlockSpec((bn, d), lambda i: (i, 0))


def _full_spec(shape):
  return pl.BlockSpec(shape, lambda i: tuple(0 for _ in shape))


def _tc_a(x, g0, b0, Wl0, bl0, Wr0):
  grid = (N // BN,)
  return pl.pallas_call(
      _tc_a_body,
      grid=grid,
      in_specs=[
          _row_spec(BN, NFEAT),
          _full_spec((NFEAT,)), _full_spec((NFEAT,)),
          _full_spec((NFEAT, NHID)), _full_spec((NHID,)),
          _full_spec((NFEAT, NHID)),
      ],
      out_specs=[_row_spec(BN, NHID), _row_spec(BN, NHID)],
      out_shape=[jax.ShapeDtypeStruct((N, NHID), jnp.float32),
                 jax.ShapeDtypeStruct((N, NHID), jnp.float32)],
  )(x, g0, b0, Wl0, bl0, Wr0)


def _tc_b(p0, p1, xr0, g1, b1, Wl1p, bl1p, Wr1p):
  grid = (N // BN,)
  return pl.pallas_call(
      _tc_b_body,
      grid=grid,
      in_specs=[
          _row_spec(BN, NHID), _row_spec(BN, NHID), _row_spec(BN, NHID),
          _full_spec((NHID,)), _full_spec((NHID,)),
          _full_spec((NHID, D1)), _full_spec((D1,)),
          _full_spec((NHID, D1)),
      ],
      out_specs=[_row_spec(BN, D1), _row_spec(BN, D1)],
      out_shape=[jax.ShapeDtypeStruct((N, D1), jnp.float32),
                 jax.ShapeDtypeStruct((N, D1), jnp.float32)],
  )(p0, p1, xr0, g1, b1, Wl1p, bl1p, Wr1p)


def _tc_c(p0, p1, xr1):
  grid = (N // BN,)
  return pl.pallas_call(
      _tc_c_body,
      grid=grid,
      in_specs=[_row_spec(BN, D1), _row_spec(BN, D1), _row_spec(BN, D1)],
      out_specs=_row_spec(BN, D1),
      out_shape=jax.ShapeDtypeStruct((N, D1), jnp.float32),
  )(p0, p1, xr1)


def kernel(x, edge_index, g0, b0, Wl0, bl0, Wr0, g1, b1, Wl1, bl1, Wr1):
  pad = EPW_PAD - EPW
  srcblk = jnp.broadcast_to(jnp.arange(pad, dtype=jnp.int32), (NW, pad))
  src = jnp.concatenate(
      [edge_index[0].astype(jnp.int32).reshape(NW, EPW), srcblk],
      axis=1).reshape(NW, NITER, CH)
  padblk = jnp.broadcast_to(N + jnp.arange(pad, dtype=jnp.int32) % (N_PAD - N),
                            (NW, pad))
  dst = jnp.concatenate(
      [edge_index[1].astype(jnp.int32).reshape(NW, EPW), padblk],
      axis=1).reshape(NW, NITER, CH)

  zero128 = jnp.zeros((ROWS_PER_TILE, NHID), jnp.float32)
  zero64 = jnp.zeros((ROWS_PER_TILE, D1), jnp.float32)

  Wl1p = jnp.zeros((NHID, D1), jnp.float32).at[:, :NCLASS].set(Wl1)
  Wr1p = jnp.zeros((NHID, D1), jnp.float32).at[:, :NCLASS].set(Wr1)
  bl1p = jnp.zeros((D1,), jnp.float32).at[:NCLASS].set(bl1)

  xl0, xr0 = _tc_a(x, g0, b0, Wl0, bl0, Wr0)
  part0 = _make_sc_agg(NHID)(xl0, src, dst, zero128)
  xl1, xr1 = _tc_b(part0[0, :N], part0[1, :N], xr0, g1, b1, Wl1p, bl1p, Wr1p)
  part1 = _make_sc_agg(D1)(xl1, src, dst, zero64)
  out = _tc_c(part1[0, :N], part1[1, :N], xr1)
  return out[:, :NCLASS]


# speedup vs baseline: 1.5145x; 1.5145x over previous
import functools

import jax
import jax.numpy as jnp
from jax import lax
from jax.experimental import pallas as pl
from jax.experimental.pallas import tpu as pltpu
from jax.experimental.pallas import tpu_sc as plsc

N = 10000
E = 320000
NFEAT = 128
NHID = 128
NCLASS = 40
D1 = 128

NC = 2
NS = 16
NW = NC * NS
EPW = E // NW
CH = 80
PC = 64
NPH = 2
NITER = PC * NPH
EPW_PAD = NITER * CH
N_PAD = 10240
ROWS_PER_TILE = N_PAD // NS
DUMMY_DST = N_PAD - 1


@functools.lru_cache(maxsize=None)
def _make_sc_agg(d):
  mesh = plsc.VectorSubcoreMesh(core_axis_name="c", subcore_axis_name="s")

  @functools.partial(
      pl.kernel,
      mesh=mesh,
      out_type=jax.ShapeDtypeStruct((NC, N_PAD, d), jnp.float32),
      scratch_types=[
          pltpu.VMEM((PC, CH), jnp.int32),
          pltpu.VMEM((PC, CH), jnp.int32),
          pltpu.VMEM((CH, d), jnp.float32),
          pltpu.VMEM((CH, d), jnp.float32),
          pltpu.VMEM_SHARED((N_PAD, d), jnp.float32),
          pltpu.SemaphoreType.DMA,
          pltpu.SemaphoreType.DMA,
      ],
  )
  def sc_agg(xl_hbm, src_hbm, dst_hbm, zero_hbm, out_hbm,
             src_v, dst_v, rows0_v, rows1_v, acc, sem0, sem1):
    c = lax.axis_index("c")
    s = lax.axis_index("s")
    wid = c * NS + s

    pltpu.sync_copy(zero_hbm, acc.at[pl.ds(s * ROWS_PER_TILE, ROWS_PER_TILE)])
    plsc.subcore_barrier()

    for p in range(NPH):
      pltpu.sync_copy(src_hbm.at[wid, pl.ds(p * PC, PC)], src_v)
      pltpu.sync_copy(dst_hbm.at[wid, pl.ds(p * PC, PC)], dst_v)

      pltpu.async_copy(xl_hbm.at[src_v.at[0]], rows0_v, sem0)
      pltpu.async_copy(xl_hbm.at[src_v.at[1]], rows1_v, sem1)

      def body(i, carry):
        j0 = 2 * i
        pltpu.make_async_copy(xl_hbm.at[src_v.at[j0]], rows0_v, sem0).wait()
        pltpu.sync_copy(rows0_v, acc.at[dst_v.at[j0]], add=True)
        pltpu.async_copy(xl_hbm.at[src_v.at[j0 + 2]], rows0_v, sem0)
        pltpu.make_async_copy(xl_hbm.at[src_v.at[j0 + 1]], rows1_v, sem1).wait()
        pltpu.sync_copy(rows1_v, acc.at[dst_v.at[j0 + 1]], add=True)
        pltpu.async_copy(xl_hbm.at[src_v.at[j0 + 3]], rows1_v, sem1)
        return carry

      lax.fori_loop(0, PC // 2 - 1, body, 0, unroll=False)

      pltpu.make_async_copy(xl_hbm.at[src_v.at[PC - 2]], rows0_v, sem0).wait()
      pltpu.sync_copy(rows0_v, acc.at[dst_v.at[PC - 2]], add=True)
      pltpu.make_async_copy(xl_hbm.at[src_v.at[PC - 1]], rows1_v, sem1).wait()
      pltpu.sync_copy(rows1_v, acc.at[dst_v.at[PC - 1]], add=True)

    plsc.subcore_barrier()

    sl = pl.ds(s * ROWS_PER_TILE, ROWS_PER_TILE)
    pltpu.sync_copy(acc.at[sl], out_hbm.at[c, sl])

  return sc_agg


BN = 1000


def _layer_norm_block(xb, g, b, eps=1e-5):
  mu = jnp.mean(xb, axis=1, keepdims=True)
  var = jnp.mean((xb - mu) ** 2, axis=1, keepdims=True)
  return (xb - mu) / jnp.sqrt(var + eps) * g + b


def _tc_a_body(x_ref, g_ref, b_ref, wl_ref, bl_ref, wr_ref, xl_ref, xr_ref):
  h = _layer_norm_block(x_ref[...], g_ref[...], b_ref[...])
  xl_ref[...] = (jnp.dot(h, wl_ref[...], preferred_element_type=jnp.float32)
                 + bl_ref[...])
  xr_ref[...] = jnp.dot(h, wr_ref[...], preferred_element_type=jnp.float32)


def _tc_b_body(p0_ref, p1_ref, xr_ref, g_ref, b_ref, wl_ref, bl_ref, wr_ref,
               xl_ref, xr1_ref):
  s = p0_ref[...] + p1_ref[...] + xr_ref[...]
  nrm = jnp.sqrt(jnp.sum(s * s, axis=1, keepdims=True))
  h = s / jnp.maximum(nrm, 1e-12)
  h = _layer_norm_block(h, g_ref[...], b_ref[...])
  h = jnp.maximum(h, 0.0)
  xl_ref[...] = (jnp.dot(h, wl_ref[...], preferred_element_type=jnp.float32)
                 + bl_ref[...])
  xr1_ref[...] = jnp.dot(h, wr_ref[...], preferred_element_type=jnp.float32)


def _tc_c_body(p0_ref, p1_ref, xr_ref, o_ref):
  s = p0_ref[...] + p1_ref[...] + xr_ref[...]
  nrm = jnp.sqrt(jnp.sum(s * s, axis=1, keepdims=True))
  h = s / jnp.maximum(nrm, 1e-12)
  col = lax.broadcasted_iota(jnp.int32, h.shape, 1)
  valid = col < NCLASS
  hm = jnp.where(valid, h, -jnp.inf)
  m = jnp.max(hm, axis=1, keepdims=True)
  ex = jnp.where(valid, jnp.exp(h - m), 0.0)
  lse = jnp.log(jnp.sum(ex, axis=1, keepdims=True))
  o_ref[...] = h - m - lse


def _row_spec(bn, d):
  return pl.BlockSpec((bn, d), lambda i: (i, 0))


def _full_spec(shape):
  return pl.BlockSpec(shape, lambda i: tuple(0 for _ in shape))


def _tc_a(x, g0, b0, Wl0, bl0, Wr0):
  grid = (N // BN,)
  return pl.pallas_call(
      _tc_a_body,
      grid=grid,
      in_specs=[
          _row_spec(BN, NFEAT),
          _full_spec((NFEAT,)), _full_spec((NFEAT,)),
          _full_spec((NFEAT, NHID)), _full_spec((NHID,)),
          _full_spec((NFEAT, NHID)),
      ],
      out_specs=[_row_spec(BN, NHID), _row_spec(BN, NHID)],
      out_shape=[jax.ShapeDtypeStruct((N, NHID), jnp.float32),
                 jax.ShapeDtypeStruct((N, NHID), jnp.float32)],
  )(x, g0, b0, Wl0, bl0, Wr0)


def _tc_b(p0, p1, xr0, g1, b1, Wl1p, bl1p, Wr1p):
  grid = (N // BN,)
  return pl.pallas_call(
      _tc_b_body,
      grid=grid,
      in_specs=[
          _row_spec(BN, NHID), _row_spec(BN, NHID), _row_spec(BN, NHID),
          _full_spec((NHID,)), _full_spec((NHID,)),
          _full_spec((NHID, D1)), _full_spec((D1,)),
          _full_spec((NHID, D1)),
      ],
      out_specs=[_row_spec(BN, D1), _row_spec(BN, D1)],
      out_shape=[jax.ShapeDtypeStruct((N, D1), jnp.float32),
                 jax.ShapeDtypeStruct((N, D1), jnp.float32)],
  )(p0, p1, xr0, g1, b1, Wl1p, bl1p, Wr1p)


def _tc_c(p0, p1, xr1):
  grid = (N // BN,)
  return pl.pallas_call(
      _tc_c_body,
      grid=grid,
      in_specs=[_row_spec(BN, D1), _row_spec(BN, D1), _row_spec(BN, D1)],
      out_specs=_row_spec(BN, D1),
      out_shape=jax.ShapeDtypeStruct((N, D1), jnp.float32),
  )(p0, p1, xr1)


def kernel(x, edge_index, g0, b0, Wl0, bl0, Wr0, g1, b1, Wl1, bl1, Wr1):
  pad = EPW_PAD - EPW
  srcblk = jnp.broadcast_to(jnp.arange(pad, dtype=jnp.int32), (NW, pad))
  src = jnp.concatenate(
      [edge_index[0].astype(jnp.int32).reshape(NW, EPW), srcblk],
      axis=1).reshape(NW, NITER, CH)
  padblk = jnp.broadcast_to(N + jnp.arange(pad, dtype=jnp.int32) % (N_PAD - N),
                            (NW, pad))
  dst = jnp.concatenate(
      [edge_index[1].astype(jnp.int32).reshape(NW, EPW), padblk],
      axis=1).reshape(NW, NITER, CH)

  zero128 = jnp.zeros((ROWS_PER_TILE, NHID), jnp.float32)
  zero64 = jnp.zeros((ROWS_PER_TILE, D1), jnp.float32)

  Wl1p = jnp.zeros((NHID, D1), jnp.float32).at[:, :NCLASS].set(Wl1)
  Wr1p = jnp.zeros((NHID, D1), jnp.float32).at[:, :NCLASS].set(Wr1)
  bl1p = jnp.zeros((D1,), jnp.float32).at[:NCLASS].set(bl1)

  xl0, xr0 = _tc_a(x, g0, b0, Wl0, bl0, Wr0)
  part0 = _make_sc_agg(NHID)(xl0, src, dst, zero128)
  xl1, xr1 = _tc_b(part0[0, :N], part0[1, :N], xr0, g1, b1, Wl1p, bl1p, Wr1p)
  part1 = _make_sc_agg(D1)(xl1, src, dst, zero64)
  out = _tc_c(part1[0, :N], part1[1, :N], xr1)
  return out[:, :NCLASS]


# speedup vs baseline: 1.7332x; 1.1444x over previous
import functools

import jax
import jax.numpy as jnp
from jax import lax
from jax.experimental import pallas as pl
from jax.experimental.pallas import tpu as pltpu
from jax.experimental.pallas import tpu_sc as plsc

N = 10000
E = 320000
NFEAT = 128
NHID = 128
NCLASS = 40
D1 = 128

NC = 2
NS = 16
NW = NC * NS
EPW = E // NW
CH = 80
PC = 64
NPH = 2
NITER = PC * NPH
EPW_PAD = NITER * CH
N_PAD = 10240
ROWS_PER_TILE = N_PAD // NS
DUMMY_DST = N_PAD - 1


@functools.lru_cache(maxsize=None)
def _make_sc_agg(d):
  mesh = plsc.VectorSubcoreMesh(core_axis_name="c", subcore_axis_name="s")

  @functools.partial(
      pl.kernel,
      mesh=mesh,
      out_type=jax.ShapeDtypeStruct((NC, N_PAD, d), jnp.float32),
      scratch_types=[
          pltpu.VMEM((PC, CH), jnp.int32),
          pltpu.VMEM((PC, CH), jnp.int32),
          pltpu.VMEM((CH, d), jnp.float32),
          pltpu.VMEM((CH, d), jnp.float32),
          pltpu.VMEM((CH, d), jnp.float32),
          pltpu.VMEM_SHARED((N_PAD, d), jnp.float32),
          pltpu.SemaphoreType.DMA,
          pltpu.SemaphoreType.DMA,
          pltpu.SemaphoreType.DMA,
      ],
  )
  def sc_agg(xl_hbm, src_hbm, dst_hbm, zero_hbm, out_hbm,
             src_v, dst_v, rows0_v, rows1_v, rows2_v, acc, sem0, sem1, sem2):
    c = lax.axis_index("c")
    s = lax.axis_index("s")
    wid = c * NS + s

    pltpu.sync_copy(zero_hbm, acc.at[pl.ds(s * ROWS_PER_TILE, ROWS_PER_TILE)])
    plsc.subcore_barrier()

    for p in range(NPH):
      pltpu.sync_copy(src_hbm.at[wid, pl.ds(p * PC, PC)], src_v)
      pltpu.sync_copy(dst_hbm.at[wid, pl.ds(p * PC, PC)], dst_v)

      bufs = ((rows0_v, sem0), (rows1_v, sem1), (rows2_v, sem2))

      def gather(j, b):
        pltpu.async_copy(xl_hbm.at[src_v.at[j]], bufs[b][0], bufs[b][1])

      def consume(j, b):
        pltpu.make_async_copy(xl_hbm.at[src_v.at[j]], bufs[b][0],
                              bufs[b][1]).wait()
        pltpu.sync_copy(bufs[b][0], acc.at[dst_v.at[j]], add=True)

      for b in range(3):
        gather(b, b)

      def body(i, carry):
        j0 = 3 * i
        for b in range(3):
          consume(j0 + b, b)
          gather(j0 + b + 3, b)
        return carry

      nfull = (PC - 4) // 3
      lax.fori_loop(0, nfull, body, 0, unroll=False)

      for j in range(3 * nfull, PC):
        b = j % 3
        consume(j, b)
        if j + 3 < PC:
          gather(j + 3, b)

    plsc.subcore_barrier()

    sl = pl.ds(s * ROWS_PER_TILE, ROWS_PER_TILE)
    pltpu.sync_copy(acc.at[sl], out_hbm.at[c, sl])

  return sc_agg


BN = 1000


def _layer_norm_block(xb, g, b, eps=1e-5):
  mu = jnp.mean(xb, axis=1, keepdims=True)
  var = jnp.mean((xb - mu) ** 2, axis=1, keepdims=True)
  return (xb - mu) / jnp.sqrt(var + eps) * g + b


def _tc_a_body(x_ref, g_ref, b_ref, wl_ref, bl_ref, wr_ref, xl_ref, xr_ref):
  h = _layer_norm_block(x_ref[...], g_ref[...], b_ref[...])
  xl_ref[...] = (jnp.dot(h, wl_ref[...], preferred_element_type=jnp.float32)
                 + bl_ref[...])
  xr_ref[...] = jnp.dot(h, wr_ref[...], preferred_element_type=jnp.float32)


def _tc_b_body(p0_ref, p1_ref, xr_ref, g_ref, b_ref, wl_ref, bl_ref, wr_ref,
               xl_ref, xr1_ref):
  s = p0_ref[...] + p1_ref[...] + xr_ref[...]
  nrm = jnp.sqrt(jnp.sum(s * s, axis=1, keepdims=True))
  h = s / jnp.maximum(nrm, 1e-12)
  h = _layer_norm_block(h, g_ref[...], b_ref[...])
  h = jnp.maximum(h, 0.0)
  xl_ref[...] = (jnp.dot(h, wl_ref[...], preferred_element_type=jnp.float32)
                 + bl_ref[...])
  xr1_ref[...] = jnp.dot(h, wr_ref[...], preferred_element_type=jnp.float32)


def _tc_c_body(p0_ref, p1_ref, xr_ref, o_ref):
  s = p0_ref[...] + p1_ref[...] + xr_ref[...]
  nrm = jnp.sqrt(jnp.sum(s * s, axis=1, keepdims=True))
  h = s / jnp.maximum(nrm, 1e-12)
  col = lax.broadcasted_iota(jnp.int32, h.shape, 1)
  valid = col < NCLASS
  hm = jnp.where(valid, h, -jnp.inf)
  m = jnp.max(hm, axis=1, keepdims=True)
  ex = jnp.where(valid, jnp.exp(h - m), 0.0)
  lse = jnp.log(jnp.sum(ex, axis=1, keepdims=True))
  o_ref[...] = h - m - lse


def _row_spec(bn, d):
  return pl.BlockSpec((bn, d), lambda i: (i, 0))


def _full_spec(shape):
  return pl.BlockSpec(shape, lambda i: tuple(0 for _ in shape))


def _tc_a(x, g0, b0, Wl0, bl0, Wr0):
  grid = (N // BN,)
  return pl.pallas_call(
      _tc_a_body,
      grid=grid,
      in_specs=[
          _row_spec(BN, NFEAT),
          _full_spec((NFEAT,)), _full_spec((NFEAT,)),
          _full_spec((NFEAT, NHID)), _full_spec((NHID,)),
          _full_spec((NFEAT, NHID)),
      ],
      out_specs=[_row_spec(BN, NHID), _row_spec(BN, NHID)],
      out_shape=[jax.ShapeDtypeStruct((N, NHID), jnp.float32),
                 jax.ShapeDtypeStruct((N, NHID), jnp.float32)],
  )(x, g0, b0, Wl0, bl0, Wr0)


def _tc_b(p0, p1, xr0, g1, b1, Wl1p, bl1p, Wr1p):
  grid = (N // BN,)
  return pl.pallas_call(
      _tc_b_body,
      grid=grid,
      in_specs=[
          _row_spec(BN, NHID), _row_spec(BN, NHID), _row_spec(BN, NHID),
          _full_spec((NHID,)), _full_spec((NHID,)),
          _full_spec((NHID, D1)), _full_spec((D1,)),
          _full_spec((NHID, D1)),
      ],
      out_specs=[_row_spec(BN, D1), _row_spec(BN, D1)],
      out_shape=[jax.ShapeDtypeStruct((N, D1), jnp.float32),
                 jax.ShapeDtypeStruct((N, D1), jnp.float32)],
  )(p0, p1, xr0, g1, b1, Wl1p, bl1p, Wr1p)


def _tc_c(p0, p1, xr1):
  grid = (N // BN,)
  return pl.pallas_call(
      _tc_c_body,
      grid=grid,
      in_specs=[_row_spec(BN, D1), _row_spec(BN, D1), _row_spec(BN, D1)],
      out_specs=_row_spec(BN, D1),
      out_shape=jax.ShapeDtypeStruct((N, D1), jnp.float32),
  )(p0, p1, xr1)


def kernel(x, edge_index, g0, b0, Wl0, bl0, Wr0, g1, b1, Wl1, bl1, Wr1):
  pad = EPW_PAD - EPW
  srcblk = jnp.broadcast_to(jnp.arange(pad, dtype=jnp.int32), (NW, pad))
  src = jnp.concatenate(
      [edge_index[0].astype(jnp.int32).reshape(NW, EPW), srcblk],
      axis=1).reshape(NW, NITER, CH)
  padblk = jnp.broadcast_to(N + jnp.arange(pad, dtype=jnp.int32) % (N_PAD - N),
                            (NW, pad))
  dst = jnp.concatenate(
      [edge_index[1].astype(jnp.int32).reshape(NW, EPW), padblk],
      axis=1).reshape(NW, NITER, CH)

  zero128 = jnp.zeros((ROWS_PER_TILE, NHID), jnp.float32)
  zero64 = jnp.zeros((ROWS_PER_TILE, D1), jnp.float32)

  Wl1p = jnp.zeros((NHID, D1), jnp.float32).at[:, :NCLASS].set(Wl1)
  Wr1p = jnp.zeros((NHID, D1), jnp.float32).at[:, :NCLASS].set(Wr1)
  bl1p = jnp.zeros((D1,), jnp.float32).at[:NCLASS].set(bl1)

  xl0, xr0 = _tc_a(x, g0, b0, Wl0, bl0, Wr0)
  part0 = _make_sc_agg(NHID)(xl0, src, dst, zero128)
  xl1, xr1 = _tc_b(part0[0, :N], part0[1, :N], xr0, g1, b1, Wl1p, bl1p, Wr1p)
  part1 = _make_sc_agg(D1)(xl1, src, dst, zero64)
  out = _tc_c(part1[0, :N], part1[1, :N], xr1)
  return out[:, :NCLASS]


# speedup vs baseline: 1.8150x; 1.0472x over previous
import functools

import jax
import jax.numpy as jnp
from jax import lax
from jax.experimental import pallas as pl
from jax.experimental.pallas import tpu as pltpu
from jax.experimental.pallas import tpu_sc as plsc

N = 10000
E = 320000
NFEAT = 128
NHID = 128
NCLASS = 40
D1 = 128

NC = 2
NS = 16
NW = NC * NS
EPW = E // NW
CH = 80
PC = 64
NPH = 2
NITER = PC * NPH
EPW_PAD = NITER * CH
N_PAD = 10240
ROWS_PER_TILE = N_PAD // NS
DUMMY_DST = N_PAD - 1


@functools.lru_cache(maxsize=None)
def _make_sc_agg(d):
  mesh = plsc.VectorSubcoreMesh(core_axis_name="c", subcore_axis_name="s")

  @functools.partial(
      pl.kernel,
      mesh=mesh,
      out_type=jax.ShapeDtypeStruct((NC, N_PAD, d), jnp.float32),
      scratch_types=[
          pltpu.VMEM((PC, CH), jnp.int32),
          pltpu.VMEM((PC, CH), jnp.int32),
          pltpu.VMEM((CH, d), jnp.float32),
          pltpu.VMEM((CH, d), jnp.float32),
          pltpu.VMEM((CH, d), jnp.float32),
          pltpu.VMEM_SHARED((N_PAD, d), jnp.float32),
          pltpu.SemaphoreType.DMA,
          pltpu.SemaphoreType.DMA,
          pltpu.SemaphoreType.DMA,
      ],
  )
  def sc_agg(xl_hbm, src_hbm, dst_hbm, zero_hbm, out_hbm,
             src_v, dst_v, rows0_v, rows1_v, rows2_v, acc, sem0, sem1, sem2):
    c = lax.axis_index("c")
    s = lax.axis_index("s")
    wid = c * NS + s

    pltpu.sync_copy(zero_hbm, acc.at[pl.ds(s * ROWS_PER_TILE, ROWS_PER_TILE)])
    plsc.subcore_barrier()

    for p in range(NPH):
      pltpu.sync_copy(src_hbm.at[wid, pl.ds(p * PC, PC)], src_v)
      pltpu.sync_copy(dst_hbm.at[wid, pl.ds(p * PC, PC)], dst_v)

      bufs = ((rows0_v, sem0), (rows1_v, sem1), (rows2_v, sem2))

      def gather(j, b):
        pltpu.async_copy(xl_hbm.at[src_v.at[j]], bufs[b][0], bufs[b][1])

      def consume(j, b):
        pltpu.make_async_copy(xl_hbm.at[src_v.at[j]], bufs[b][0],
                              bufs[b][1]).wait()
        pltpu.sync_copy(bufs[b][0], acc.at[dst_v.at[j]], add=True)

      for b in range(3):
        gather(b, b)

      def body(i, carry):
        j0 = 3 * i
        for b in range(3):
          consume(j0 + b, b)
          gather(j0 + b + 3, b)
        return carry

      nfull = (PC - 4) // 3
      lax.fori_loop(0, nfull, body, 0, unroll=False)

      for j in range(3 * nfull, PC):
        b = j % 3
        consume(j, b)
        if j + 3 < PC:
          gather(j + 3, b)

    plsc.subcore_barrier()

    sl = pl.ds(s * ROWS_PER_TILE, ROWS_PER_TILE)
    pltpu.sync_copy(acc.at[sl], out_hbm.at[c, sl])

  return sc_agg


BN = 1000


def _layer_norm_block(xb, g, b, eps=1e-5):
  mu = jnp.mean(xb, axis=1, keepdims=True)
  var = jnp.mean((xb - mu) ** 2, axis=1, keepdims=True)
  return (xb - mu) / jnp.sqrt(var + eps) * g + b


def _tc_a_body(x_ref, g_ref, b_ref, wl_ref, bl_ref, wr_ref, xl_ref, xr_ref):
  h = _layer_norm_block(x_ref[...], g_ref[...], b_ref[...])
  xl_ref[...] = (jnp.dot(h, wl_ref[...], preferred_element_type=jnp.float32)
                 + bl_ref[...])
  xr_ref[...] = jnp.dot(h, wr_ref[...], preferred_element_type=jnp.float32)


def _tc_b_body(p0_ref, p1_ref, xr_ref, g_ref, b_ref, wl_ref, bl_ref, wr_ref,
               xl_ref, xr1_ref):
  s = p0_ref[0] + p1_ref[0] + xr_ref[...]
  nrm = jnp.sqrt(jnp.sum(s * s, axis=1, keepdims=True))
  h = s / jnp.maximum(nrm, 1e-12)
  h = _layer_norm_block(h, g_ref[...], b_ref[...])
  h = jnp.maximum(h, 0.0)
  xl_ref[...] = (jnp.dot(h, wl_ref[...], preferred_element_type=jnp.float32)
                 + bl_ref[...])
  xr1_ref[...] = jnp.dot(h, wr_ref[...], preferred_element_type=jnp.float32)


def _tc_c_body(p0_ref, p1_ref, xr_ref, o_ref):
  s = p0_ref[0] + p1_ref[0] + xr_ref[...]
  nrm = jnp.sqrt(jnp.sum(s * s, axis=1, keepdims=True))
  h = s / jnp.maximum(nrm, 1e-12)
  col = lax.broadcasted_iota(jnp.int32, h.shape, 1)
  valid = col < NCLASS
  hm = jnp.where(valid, h, -jnp.inf)
  m = jnp.max(hm, axis=1, keepdims=True)
  ex = jnp.where(valid, jnp.exp(h - m), 0.0)
  lse = jnp.log(jnp.sum(ex, axis=1, keepdims=True))
  o_ref[...] = h - m - lse


def _row_spec(bn, d):
  return pl.BlockSpec((bn, d), lambda i: (i, 0))


def _part_spec(bn, d, c):
  return pl.BlockSpec((1, bn, d), lambda i, _c=c: (_c, i, 0))


def _full_spec(shape):
  return pl.BlockSpec(shape, lambda i: tuple(0 for _ in shape))


def _tc_a(x, g0, b0, Wl0, bl0, Wr0):
  grid = (N // BN,)
  return pl.pallas_call(
      _tc_a_body,
      grid=grid,
      in_specs=[
          _row_spec(BN, NFEAT),
          _full_spec((NFEAT,)), _full_spec((NFEAT,)),
          _full_spec((NFEAT, NHID)), _full_spec((NHID,)),
          _full_spec((NFEAT, NHID)),
      ],
      out_specs=[_row_spec(BN, NHID), _row_spec(BN, NHID)],
      out_shape=[jax.ShapeDtypeStruct((N, NHID), jnp.float32),
                 jax.ShapeDtypeStruct((N, NHID), jnp.float32)],
  )(x, g0, b0, Wl0, bl0, Wr0)


def _tc_b(part, xr0, g1, b1, Wl1p, bl1p, Wr1p):
  grid = (N // BN,)
  return pl.pallas_call(
      _tc_b_body,
      grid=grid,
      in_specs=[
          _part_spec(BN, NHID, 0), _part_spec(BN, NHID, 1), _row_spec(BN, NHID),
          _full_spec((NHID,)), _full_spec((NHID,)),
          _full_spec((NHID, D1)), _full_spec((D1,)),
          _full_spec((NHID, D1)),
      ],
      out_specs=[_row_spec(BN, D1), _row_spec(BN, D1)],
      out_shape=[jax.ShapeDtypeStruct((N, D1), jnp.float32),
                 jax.ShapeDtypeStruct((N, D1), jnp.float32)],
  )(part, part, xr0, g1, b1, Wl1p, bl1p, Wr1p)


def _tc_c(part, xr1):
  grid = (N // BN,)
  return pl.pallas_call(
      _tc_c_body,
      grid=grid,
      in_specs=[_part_spec(BN, D1, 0), _part_spec(BN, D1, 1),
                _row_spec(BN, D1)],
      out_specs=_row_spec(BN, D1),
      out_shape=jax.ShapeDtypeStruct((N, D1), jnp.float32),
  )(part, part, xr1)


def kernel(x, edge_index, g0, b0, Wl0, bl0, Wr0, g1, b1, Wl1, bl1, Wr1):
  pad = EPW_PAD - EPW
  srcblk = jnp.broadcast_to(jnp.arange(pad, dtype=jnp.int32), (NW, pad))
  src = jnp.concatenate(
      [edge_index[0].astype(jnp.int32).reshape(NW, EPW), srcblk],
      axis=1).reshape(NW, NITER, CH)
  padblk = jnp.broadcast_to(N + jnp.arange(pad, dtype=jnp.int32) % (N_PAD - N),
                            (NW, pad))
  dst = jnp.concatenate(
      [edge_index[1].astype(jnp.int32).reshape(NW, EPW), padblk],
      axis=1).reshape(NW, NITER, CH)

  zero128 = jnp.zeros((ROWS_PER_TILE, NHID), jnp.float32)
  zero64 = jnp.zeros((ROWS_PER_TILE, D1), jnp.float32)

  Wl1p = jnp.zeros((NHID, D1), jnp.float32).at[:, :NCLASS].set(Wl1)
  Wr1p = jnp.zeros((NHID, D1), jnp.float32).at[:, :NCLASS].set(Wr1)
  bl1p = jnp.zeros((D1,), jnp.float32).at[:NCLASS].set(bl1)

  xl0, xr0 = _tc_a(x, g0, b0, Wl0, bl0, Wr0)
  part0 = _make_sc_agg(NHID)(xl0, src, dst, zero128)
  xl1, xr1 = _tc_b(part0, xr0, g1, b1, Wl1p, bl1p, Wr1p)
  part1 = _make_sc_agg(D1)(xl1, src, dst, zero64)
  out = _tc_c(part1, xr1)
  return out[:, :NCLASS]


# speedup vs baseline: 1.8481x; 1.0182x over previous
import functools

import jax
import jax.numpy as jnp
from jax import lax
from jax.experimental import pallas as pl
from jax.experimental.pallas import tpu as pltpu
from jax.experimental.pallas import tpu_sc as plsc

N = 10000
E = 320000
NFEAT = 128
NHID = 128
NCLASS = 40
D1 = 128

NC = 2
NS = 16
NW = NC * NS
EPW = E // NW
CH = 80
NITER = EPW // CH
PHASES = ((0, 64), (64, 61))
N_PAD = 10240
ROWS_PER_TILE = N_PAD // NS


@functools.lru_cache(maxsize=None)
def _make_sc_agg(d):
  mesh = plsc.VectorSubcoreMesh(core_axis_name="c", subcore_axis_name="s")

  @functools.partial(
      pl.kernel,
      mesh=mesh,
      out_type=jax.ShapeDtypeStruct((NC, N_PAD, d), jnp.float32),
      scratch_types=[
          pltpu.VMEM((PHASES[0][1], CH), jnp.int32),
          pltpu.VMEM((PHASES[0][1], CH), jnp.int32),
          pltpu.VMEM((CH, d), jnp.float32),
          pltpu.VMEM((CH, d), jnp.float32),
          pltpu.VMEM((CH, d), jnp.float32),
          pltpu.VMEM_SHARED((N_PAD, d), jnp.float32),
          pltpu.SemaphoreType.DMA,
          pltpu.SemaphoreType.DMA,
          pltpu.SemaphoreType.DMA,
      ],
  )
  def sc_agg(xl_hbm, src_hbm, dst_hbm, zero_hbm, out_hbm,
             src_v, dst_v, rows0_v, rows1_v, rows2_v, acc, sem0, sem1, sem2):
    c = lax.axis_index("c")
    s = lax.axis_index("s")
    wid = c * NS + s

    pltpu.sync_copy(zero_hbm, acc.at[pl.ds(s * ROWS_PER_TILE, ROWS_PER_TILE)])
    plsc.subcore_barrier()

    bufs = ((rows0_v, sem0), (rows1_v, sem1), (rows2_v, sem2))

    def gather(j, b):
      pltpu.async_copy(xl_hbm.at[src_v.at[j]], bufs[b][0], bufs[b][1])

    def consume(j, b):
      pltpu.make_async_copy(xl_hbm.at[src_v.at[j]], bufs[b][0],
                            bufs[b][1]).wait()
      pltpu.sync_copy(bufs[b][0], acc.at[dst_v.at[j]], add=True)

    for base, cnt in PHASES:
      pltpu.sync_copy(src_hbm.at[wid, pl.ds(base, cnt)],
                      src_v.at[pl.ds(0, cnt)])
      pltpu.sync_copy(dst_hbm.at[wid, pl.ds(base, cnt)],
                      dst_v.at[pl.ds(0, cnt)])

      for b in range(3):
        gather(b, b)

      def body(i, carry):
        j0 = 3 * i
        for b in range(3):
          consume(j0 + b, b)
          gather(j0 + b + 3, b)
        return carry

      nfull = (cnt - 4) // 3
      lax.fori_loop(0, nfull, body, 0, unroll=False)

      for j in range(3 * nfull, cnt):
        b = j % 3
        consume(j, b)
        if j + 3 < cnt:
          gather(j + 3, b)

    plsc.subcore_barrier()

    sl = pl.ds(s * ROWS_PER_TILE, ROWS_PER_TILE)
    pltpu.sync_copy(acc.at[sl], out_hbm.at[c, sl])

  return sc_agg


BN = 1000


def _layer_norm_block(xb, g, b, eps=1e-5):
  mu = jnp.mean(xb, axis=1, keepdims=True)
  var = jnp.mean((xb - mu) ** 2, axis=1, keepdims=True)
  return (xb - mu) / jnp.sqrt(var + eps) * g + b


def _tc_a_body(x_ref, g_ref, b_ref, wl_ref, bl_ref, wr_ref, xl_ref, xr_ref):
  h = _layer_norm_block(x_ref[...], g_ref[...], b_ref[...])
  xl_ref[...] = (jnp.dot(h, wl_ref[...], preferred_element_type=jnp.float32)
                 + bl_ref[...])
  xr_ref[...] = jnp.dot(h, wr_ref[...], preferred_element_type=jnp.float32)


def _tc_b_body(p0_ref, p1_ref, xr_ref, g_ref, b_ref, wl_ref, bl_ref, wr_ref,
               xl_ref, xr1_ref):
  s = p0_ref[0] + p1_ref[0] + xr_ref[...]
  nrm = jnp.sqrt(jnp.sum(s * s, axis=1, keepdims=True))
  h = s / jnp.maximum(nrm, 1e-12)
  h = _layer_norm_block(h, g_ref[...], b_ref[...])
  h = jnp.maximum(h, 0.0)
  xl_ref[...] = (jnp.dot(h, wl_ref[...], preferred_element_type=jnp.float32)
                 + bl_ref[...])
  xr1_ref[...] = jnp.dot(h, wr_ref[...], preferred_element_type=jnp.float32)


def _tc_c_body(p0_ref, p1_ref, xr_ref, o_ref):
  s = (p0_ref[0] + p1_ref[0] + xr_ref[...])[:, :NCLASS]
  nrm = jnp.sqrt(jnp.sum(s * s, axis=1, keepdims=True))
  h = s / jnp.maximum(nrm, 1e-12)
  m = jnp.max(h, axis=1, keepdims=True)
  lse = jnp.log(jnp.sum(jnp.exp(h - m), axis=1, keepdims=True))
  o_ref[...] = h - m - lse


def _row_spec(bn, d):
  return pl.BlockSpec((bn, d), lambda i: (i, 0))


def _part_spec(bn, d, c):
  return pl.BlockSpec((1, bn, d), lambda i, _c=c: (_c, i, 0))


def _full_spec(shape):
  return pl.BlockSpec(shape, lambda i: tuple(0 for _ in shape))


def _tc_a(x, g0, b0, Wl0, bl0, Wr0):
  grid = (N // BN,)
  return pl.pallas_call(
      _tc_a_body,
      grid=grid,
      in_specs=[
          _row_spec(BN, NFEAT),
          _full_spec((NFEAT,)), _full_spec((NFEAT,)),
          _full_spec((NFEAT, NHID)), _full_spec((NHID,)),
          _full_spec((NFEAT, NHID)),
      ],
      out_specs=[_row_spec(BN, NHID), _row_spec(BN, NHID)],
      out_shape=[jax.ShapeDtypeStruct((N, NHID), jnp.float32),
                 jax.ShapeDtypeStruct((N, NHID), jnp.float32)],
  )(x, g0, b0, Wl0, bl0, Wr0)


def _tc_b(part, xr0, g1, b1, Wl1p, bl1p, Wr1p):
  grid = (N // BN,)
  return pl.pallas_call(
      _tc_b_body,
      grid=grid,
      in_specs=[
          _part_spec(BN, NHID, 0), _part_spec(BN, NHID, 1), _row_spec(BN, NHID),
          _full_spec((NHID,)), _full_spec((NHID,)),
          _full_spec((NHID, D1)), _full_spec((D1,)),
          _full_spec((NHID, D1)),
      ],
      out_specs=[_row_spec(BN, D1), _row_spec(BN, D1)],
      out_shape=[jax.ShapeDtypeStruct((N, D1), jnp.float32),
                 jax.ShapeDtypeStruct((N, D1), jnp.float32)],
  )(part, part, xr0, g1, b1, Wl1p, bl1p, Wr1p)


def _tc_c(part, xr1):
  grid = (N // BN,)
  return pl.pallas_call(
      _tc_c_body,
      grid=grid,
      in_specs=[_part_spec(BN, D1, 0), _part_spec(BN, D1, 1),
                _row_spec(BN, D1)],
      out_specs=_row_spec(BN, NCLASS),
      out_shape=jax.ShapeDtypeStruct((N, NCLASS), jnp.float32),
  )(part, part, xr1)


def kernel(x, edge_index, g0, b0, Wl0, bl0, Wr0, g1, b1, Wl1, bl1, Wr1):
  src = edge_index[0].astype(jnp.int32).reshape(NW, NITER, CH)
  dst = edge_index[1].astype(jnp.int32).reshape(NW, NITER, CH)

  zero128 = jnp.zeros((ROWS_PER_TILE, NHID), jnp.float32)
  zero64 = jnp.zeros((ROWS_PER_TILE, D1), jnp.float32)

  Wl1p = jnp.zeros((NHID, D1), jnp.float32).at[:, :NCLASS].set(Wl1)
  Wr1p = jnp.zeros((NHID, D1), jnp.float32).at[:, :NCLASS].set(Wr1)
  bl1p = jnp.zeros((D1,), jnp.float32).at[:NCLASS].set(bl1)

  xl0, xr0 = _tc_a(x, g0, b0, Wl0, bl0, Wr0)
  part0 = _make_sc_agg(NHID)(xl0, src, dst, zero128)
  xl1, xr1 = _tc_b(part0, xr0, g1, b1, Wl1p, bl1p, Wr1p)
  part1 = _make_sc_agg(D1)(xl1, src, dst, zero64)
  return _tc_c(part1, xr1)
